# Initial kernel scaffold; baseline (speedup 1.0000x reference)
import jax
import jax.numpy as jnp
from jax.experimental import pallas as pl


def kernel(x, edge_index, batch, params, head):
    raise NotImplementedError("write your pallas kernel here")


# speedup vs baseline: 2.6212x; 2.6212x over previous
import functools

import jax
import jax.numpy as jnp
from jax import lax
from jax.experimental import pallas as pl
from jax.experimental.pallas import tpu as pltpu
from jax.experimental.pallas import tpu_sc as plsc

N = 50000
E = 800000
H = 128
NSUB = 16
NCORE = 2
G = 128

BLK = 12512
TRROWS = 32
SROWS = BLK + TRROWS
RPS = SROWS // NSUB
NPAD = 4 * BLK

EPW = E // NSUB
NCH = EPW // G
TAIL = EPW - NCH * G

f32 = jnp.float32
i32 = jnp.int32


def _agg_body(t_hbm, srcr, dstr, out, aggsh, zrows, sv, dv,
              fifo_s, fifo_d, rows, gsem):
    c = lax.axis_index("c")
    s = lax.axis_index("s")
    lanes = lax.iota(i32, 16)

    def zf(j, _):
        for k in range(8):
            zrows[j, pl.ds(k * 16, 16)] = jnp.zeros((16,), f32)
        return 0
    lax.fori_loop(0, 56, zf, 0)

    def fire():
        pltpu.async_copy(t_hbm.at[fifo_s.at[0]], rows, gsem).wait()
        pltpu.sync_copy(rows, aggsh.at[fifo_d.at[0]], add=True)
        for k in range(8):
            sl = pl.ds(k * 16, 16)
            fifo_d[0, sl] = fifo_d[1, sl]
            fifo_s[0, sl] = fifo_s[1, sl]

    def scan_groups(cnt, lo, ngroups):
        base = jnp.full((16,), cnt, i32)
        for j in range(ngroups):
            sl = pl.ds(j * 16, 16)
            d = dv[sl]
            sr = sv[sl]
            dr = d - lo
            m = (dr >= 0) & (dr < BLK)
            csum = plsc.cumsum(m.astype(i32))
            pos = base + csum - 1
            row = (pos >= G).astype(i32)
            col = pos - row * G
            plsc.store_scatter(fifo_d, [row, col], dr, mask=m)
            plsc.store_scatter(fifo_s, [row, col], sr, mask=m)
            base = base + plsc.all_reduce_population_count(m)
        cnt = jnp.max(base)
        for _ in range(2):
            fired = cnt >= G
            pl.when(fired)(fire)
            cnt = cnt - jnp.where(fired, G, 0)
        return cnt

    for p in range(2):
        b = c * 2 + p
        lo = b * BLK

        def zb(j, _):
            pltpu.sync_copy(zrows, aggsh.at[pl.ds(s * RPS + j * 56, 56)])
            return 0
        lax.fori_loop(0, 14, zb, 0)
        plsc.subcore_barrier()

        e0 = s * EPW

        def chunk(i, cnt):
            pltpu.sync_copy(srcr.at[pl.ds(e0 + i * G, G)], sv)
            pltpu.sync_copy(dstr.at[pl.ds(e0 + i * G, G)], dv)
            return scan_groups(cnt, lo, 8)
        cnt = lax.fori_loop(0, NCH, chunk, jnp.int32(0))

        pltpu.sync_copy(srcr.at[pl.ds(e0 + NCH * G, TAIL)],
                        sv.at[pl.ds(0, TAIL)])
        pltpu.sync_copy(dstr.at[pl.ds(e0 + NCH * G, TAIL)],
                        dv.at[pl.ds(0, TAIL)])
        cnt = scan_groups(cnt, lo, TAIL // 16)

        cnt_v = jnp.full((16,), cnt, i32)
        padsrc = (s * 16 + lanes) * 8
        for k in range(8):
            sl = pl.ds(k * 16, 16)
            padmask = (lanes + k * 16) >= cnt_v
            fifo_d[0, sl] = jnp.where(padmask, BLK + (lanes & 15), fifo_d[0, sl])
            fifo_s[0, sl] = jnp.where(padmask, padsrc, fifo_s[0, sl])
        pl.when(cnt > 0)(fire)
        plsc.subcore_barrier()

        @pl.when(s < NSUB - 1)
        def _():
            pltpu.sync_copy(aggsh.at[pl.ds(s * RPS, RPS)],
                            out.at[pl.ds(lo + s * RPS, RPS)])

        @pl.when(s == NSUB - 1)
        def _():
            pltpu.sync_copy(aggsh.at[pl.ds((NSUB - 1) * RPS, BLK - (NSUB - 1) * RPS)],
                            out.at[pl.ds(lo + (NSUB - 1) * RPS, BLK - (NSUB - 1) * RPS)])

        plsc.subcore_barrier()


_AGG_CACHE = {}


def _get_agg():
    if "agg" not in _AGG_CACHE:
        mesh = plsc.VectorSubcoreMesh(core_axis_name="c", subcore_axis_name="s",
                                      num_cores=NCORE, num_subcores=NSUB)
        _AGG_CACHE["agg"] = functools.partial(
            pl.kernel,
            out_type=jax.ShapeDtypeStruct((NPAD, H), f32),
            mesh=mesh,
            compiler_params=pltpu.CompilerParams(needs_layout_passes=False),
            scratch_types=[
                pltpu.VMEM_SHARED((SROWS, H), f32),
                pltpu.VMEM((56, H), f32),
                pltpu.VMEM((G,), i32),
                pltpu.VMEM((G,), i32),
                pltpu.VMEM((2, G), i32),
                pltpu.VMEM((2, G), i32),
                pltpu.VMEM((G, H), f32),
                pltpu.SemaphoreType.DMA,
            ],
        )(_agg_body)
    return _AGG_CACHE["agg"]


BN = 2048
NBLK = (N + BN - 1) // BN


def _pre_body(x_ref, w1, o_ref):
    o_ref[...] = jnp.dot(x_ref[...], w1[...], preferred_element_type=f32)


def _mlp_body(t_ref, s_ref, b1, w2, b2, w3, b3, w1n, o_ref):
    z = jax.nn.relu(t_ref[...] + s_ref[...] + b1[...])
    z = jax.nn.relu(jnp.dot(z, w2[...], preferred_element_type=f32) + b2[...])
    z = jnp.dot(z, w3[...], preferred_element_type=f32) + b3[...]
    h = jax.nn.relu(z)
    o_ref[...] = jnp.dot(h, w1n[...], preferred_element_type=f32)


def _mlp_last_body(t_ref, s_ref, b1, w2, b2, w3, b3, o_ref):
    z = jax.nn.relu(t_ref[...] + s_ref[...] + b1[...])
    z = jax.nn.relu(jnp.dot(z, w2[...], preferred_element_type=f32) + b2[...])
    z = jnp.dot(z, w3[...], preferred_element_type=f32) + b3[...]
    o_ref[...] = jax.nn.relu(z)


_pre = pl.pallas_call(
    _pre_body,
    grid=(NBLK,),
    in_specs=[
        pl.BlockSpec((BN, 4), lambda i: (i, 0)),
        pl.BlockSpec((4, H), lambda i: (0, 0)),
    ],
    out_specs=pl.BlockSpec((BN, H), lambda i: (i, 0)),
    out_shape=jax.ShapeDtypeStruct((N, H), f32),
)


def _make_mlp(last):
    body = _mlp_last_body if last else _mlp_body
    wspecs = [pl.BlockSpec((1, H), lambda i: (0, 0)),
              pl.BlockSpec((H, H), lambda i: (0, 0)),
              pl.BlockSpec((1, H), lambda i: (0, 0)),
              pl.BlockSpec((H, H), lambda i: (0, 0)),
              pl.BlockSpec((1, H), lambda i: (0, 0))]
    if not last:
        wspecs.append(pl.BlockSpec((H, H), lambda i: (0, 0)))
    return pl.pallas_call(
        body,
        grid=(NBLK,),
        in_specs=[
            pl.BlockSpec((BN, H), lambda i: (i, 0)),
            pl.BlockSpec((BN, H), lambda i: (i, 0)),
        ] + wspecs,
        out_specs=pl.BlockSpec((BN, H), lambda i: (i, 0)),
        out_shape=jax.ShapeDtypeStruct((N, H), f32),
    )


_mlp_mid = _make_mlp(False)
_mlp_last = _make_mlp(True)

NG = 64


def _pool_head_body(h_ref, b_ref, wh1, bh1, wh2, bh2, o_ref, sacc, cacc):
    i = pl.program_id(0)

    @pl.when(i == 0)
    def _():
        sacc[...] = jnp.zeros((NG, H), f32)
        cacc[...] = jnp.zeros((NG, 1), f32)

    bv = b_ref[0]
    oh = (bv == lax.broadcasted_iota(i32, (NG, BN), 0)).astype(f32)
    rowid = lax.broadcasted_iota(i32, (BN, H), 0) + i * BN
    hv = jnp.where(rowid < N, h_ref[...], 0.0)
    sacc[...] += jnp.dot(oh, hv, preferred_element_type=f32)
    cacc[...] += jnp.dot(oh, jnp.ones((BN, 1), f32), preferred_element_type=f32)

    @pl.when(i == NBLK - 1)
    def _():
        hg = sacc[...] / jnp.maximum(cacc[...], 1.0)
        y = jax.nn.relu(jnp.dot(hg, wh1[...], preferred_element_type=f32)
                        + bh1[...])
        o_ref[...] = jnp.dot(y, wh2[...], preferred_element_type=f32) + bh2[...]


_pool_head = pl.pallas_call(
    _pool_head_body,
    grid=(NBLK,),
    in_specs=[
        pl.BlockSpec((BN, H), lambda i: (i, 0)),
        pl.BlockSpec((1, 1, BN), lambda i: (i, 0, 0)),
        pl.BlockSpec((H, H), lambda i: (0, 0)),
        pl.BlockSpec((1, H), lambda i: (0, 0)),
        pl.BlockSpec((H, 1), lambda i: (0, 0)),
        pl.BlockSpec((1, 1), lambda i: (0, 0)),
    ],
    out_specs=pl.BlockSpec((NG, 1), lambda i: (0, 0)),
    out_shape=jax.ShapeDtypeStruct((NG, 1), f32),
    scratch_shapes=[
        pltpu.VMEM((NG, H), f32),
        pltpu.VMEM((NG, 1), f32),
    ],
)


def kernel(x, edge_index, batch, params, head):
    src = edge_index[0].astype(i32)
    dst = edge_index[1].astype(i32)

    agg = _get_agg()
    t = _pre(x.astype(f32), params[0][0][0])
    for k in range(4):
        (_, b1), (w2, b2), (w3, b3) = params[k]
        s = agg(t, src, dst)
        if k < 3:
            t = _mlp_mid(t, s, b1.reshape(1, H), w2, b2.reshape(1, H),
                         w3, b3.reshape(1, H), params[k + 1][0][0])
        else:
            h = _mlp_last(t, s, b1.reshape(1, H), w2, b2.reshape(1, H),
                          w3, b3.reshape(1, H))

    (wh1, bh1), (wh2, bh2) = head
    batch_p = jnp.pad(batch.astype(i32), (0, NBLK * BN - N),
                      constant_values=NG).reshape(NBLK, 1, BN)
    out = _pool_head(h, batch_p, wh1, bh1.reshape(1, H),
                     wh2, bh2.reshape(1, 1))
    return out.reshape(-1)


# speedup vs baseline: 5.0937x; 1.9433x over previous
import functools

import jax
import jax.numpy as jnp
from jax import lax
from jax.experimental import pallas as pl
from jax.experimental.pallas import tpu as pltpu
from jax.experimental.pallas import tpu_sc as plsc

N = 50000
E = 800000
H = 128
NSUB = 16
NCORE = 2
G = 128

BLK = 12512
TRROWS = 32
SROWS = BLK + TRROWS
RPS = SROWS // NSUB
NPAD = 4 * BLK

EPW = E // NSUB
CH = 2000
NCH = EPW // CH
SUB = 80
NSC = CH // SUB

f32 = jnp.float32
i32 = jnp.int32


def _agg_body(t_hbm, srcr, dstr, out, aggsh, zrows, sv, dv,
              fifo_s, fifo_d, rows, gsem):
    c = lax.axis_index("c")
    s = lax.axis_index("s")
    lanes = lax.iota(i32, 16)

    def zf(j, _):
        for k in range(8):
            zrows[j, pl.ds(k * 16, 16)] = jnp.zeros((16,), f32)
        return 0
    lax.fori_loop(0, 8, zf, 0)

    def fire():
        pltpu.async_copy(t_hbm.at[fifo_s.at[0]], rows, gsem).wait()
        pltpu.sync_copy(rows, aggsh.at[fifo_d.at[0]], add=True)
        for k in range(8):
            sl = pl.ds(k * 16, 16)
            fifo_d[0, sl] = fifo_d[1, sl]
            fifo_s[0, sl] = fifo_s[1, sl]

    def scan_sub(cnt, lo, off):
        base = jnp.full((16,), cnt, i32)
        for j in range(SUB // 16):
            sl = pl.ds(off + j * 16, 16)
            d = dv[sl]
            sr = sv[sl]
            dr = d - lo
            m = (dr >= 0) & (dr < BLK)
            csum = plsc.cumsum(m.astype(i32))
            pos = base + csum - 1
            row = (pos >= G).astype(i32)
            col = pos - row * G
            plsc.store_scatter(fifo_d, [row, col], dr, mask=m)
            plsc.store_scatter(fifo_s, [row, col], sr, mask=m)
            base = base + plsc.all_reduce_population_count(m)
        cnt = jnp.max(base)
        fired = cnt >= G
        pl.when(fired)(fire)
        return cnt - jnp.where(fired, G, 0)

    for p in range(2):
        b = c * 2 + p
        lo = b * BLK

        def zb(j, _):
            pltpu.sync_copy(zrows, aggsh.at[pl.ds(s * RPS + j * 8, 8)])
            return 0
        lax.fori_loop(0, RPS // 8, zb, 0)
        plsc.subcore_barrier()

        e0 = s * EPW

        def chunk(i, cnt):
            pltpu.sync_copy(srcr.at[pl.ds(e0 + i * CH, CH)], sv)
            pltpu.sync_copy(dstr.at[pl.ds(e0 + i * CH, CH)], dv)

            def sub(g, cnt):
                return scan_sub(cnt, lo, g * SUB)
            return lax.fori_loop(0, NSC, sub, cnt)
        cnt = lax.fori_loop(0, NCH, chunk, jnp.int32(0))

        cnt_v = jnp.full((16,), cnt, i32)
        padsrc = (s * 16 + lanes) * 8
        for k in range(8):
            sl = pl.ds(k * 16, 16)
            padmask = (lanes + k * 16) >= cnt_v
            fifo_d[0, sl] = jnp.where(padmask, BLK + (lanes & 15), fifo_d[0, sl])
            fifo_s[0, sl] = jnp.where(padmask, padsrc, fifo_s[0, sl])
        pl.when(cnt > 0)(fire)
        plsc.subcore_barrier()

        @pl.when(s < NSUB - 1)
        def _():
            pltpu.sync_copy(aggsh.at[pl.ds(s * RPS, RPS)],
                            out.at[pl.ds(lo + s * RPS, RPS)])

        @pl.when(s == NSUB - 1)
        def _():
            pltpu.sync_copy(aggsh.at[pl.ds((NSUB - 1) * RPS, BLK - (NSUB - 1) * RPS)],
                            out.at[pl.ds(lo + (NSUB - 1) * RPS, BLK - (NSUB - 1) * RPS)])

        plsc.subcore_barrier()


_AGG_CACHE = {}


def _get_agg():
    if "agg" not in _AGG_CACHE:
        mesh = plsc.VectorSubcoreMesh(core_axis_name="c", subcore_axis_name="s",
                                      num_cores=NCORE, num_subcores=NSUB)
        _AGG_CACHE["agg"] = functools.partial(
            pl.kernel,
            out_type=jax.ShapeDtypeStruct((NPAD, H), f32),
            mesh=mesh,
            compiler_params=pltpu.CompilerParams(needs_layout_passes=False),
            scratch_types=[
                pltpu.VMEM_SHARED((SROWS, H), f32),
                pltpu.VMEM((8, H), f32),
                pltpu.VMEM((CH,), i32),
                pltpu.VMEM((CH,), i32),
                pltpu.VMEM((2, G), i32),
                pltpu.VMEM((2, G), i32),
                pltpu.VMEM((G, H), f32),
                pltpu.SemaphoreType.DMA,
            ],
        )(_agg_body)
    return _AGG_CACHE["agg"]


BN = 2048
NBLK = (N + BN - 1) // BN


def _mlp_body(h_ref, s_ref, w1, b1, w2, b2, w3, b3, o_ref):
    u = h_ref[...] + s_ref[...]
    z = jax.nn.relu(jnp.dot(u, w1[...], preferred_element_type=f32) + b1[...])
    z = jax.nn.relu(jnp.dot(z, w2[...], preferred_element_type=f32) + b2[...])
    z = jnp.dot(z, w3[...], preferred_element_type=f32) + b3[...]
    o_ref[...] = jax.nn.relu(z)


_mlp = pl.pallas_call(
    _mlp_body,
    grid=(NBLK,),
    in_specs=[
        pl.BlockSpec((BN, H), lambda i: (i, 0)),
        pl.BlockSpec((BN, H), lambda i: (i, 0)),
        pl.BlockSpec((H, H), lambda i: (0, 0)),
        pl.BlockSpec((1, H), lambda i: (0, 0)),
        pl.BlockSpec((H, H), lambda i: (0, 0)),
        pl.BlockSpec((1, H), lambda i: (0, 0)),
        pl.BlockSpec((H, H), lambda i: (0, 0)),
        pl.BlockSpec((1, H), lambda i: (0, 0)),
    ],
    out_specs=pl.BlockSpec((BN, H), lambda i: (i, 0)),
    out_shape=jax.ShapeDtypeStruct((N, H), f32),
)

NG = 64


def _pool_head_body(h_ref, b_ref, wh1, bh1, wh2, bh2, o_ref, sacc, cacc):
    i = pl.program_id(0)

    @pl.when(i == 0)
    def _():
        sacc[...] = jnp.zeros((NG, H), f32)
        cacc[...] = jnp.zeros((NG, 1), f32)

    bv = b_ref[0]
    oh = (bv == lax.broadcasted_iota(i32, (NG, BN), 0)).astype(f32)
    rowid = lax.broadcasted_iota(i32, (BN, H), 0) + i * BN
    hv = jnp.where(rowid < N, h_ref[...], 0.0)
    sacc[...] += jnp.dot(oh, hv, preferred_element_type=f32,
                         precision=lax.Precision.HIGHEST)
    cacc[...] += jnp.dot(oh, jnp.ones((BN, 1), f32), preferred_element_type=f32)

    @pl.when(i == NBLK - 1)
    def _():
        hg = sacc[...] / jnp.maximum(cacc[...], 1.0)
        y = jax.nn.relu(jnp.dot(hg, wh1[...], preferred_element_type=f32)
                        + bh1[...])
        o_ref[...] = jnp.dot(y, wh2[...], preferred_element_type=f32) + bh2[...]


_pool_head = pl.pallas_call(
    _pool_head_body,
    grid=(NBLK,),
    in_specs=[
        pl.BlockSpec((BN, H), lambda i: (i, 0)),
        pl.BlockSpec((1, 1, BN), lambda i: (i, 0, 0)),
        pl.BlockSpec((H, H), lambda i: (0, 0)),
        pl.BlockSpec((1, H), lambda i: (0, 0)),
        pl.BlockSpec((H, 1), lambda i: (0, 0)),
        pl.BlockSpec((1, 1), lambda i: (0, 0)),
    ],
    out_specs=pl.BlockSpec((NG, 1), lambda i: (0, 0)),
    out_shape=jax.ShapeDtypeStruct((NG, 1), f32),
    scratch_shapes=[
        pltpu.VMEM((NG, H), f32),
        pltpu.VMEM((NG, 1), f32),
    ],
)


def kernel(x, edge_index, batch, params, head):
    src = edge_index[0].astype(i32)
    dst = edge_index[1].astype(i32)

    agg = _get_agg()
    h = jnp.pad(x.astype(f32), ((0, 0), (0, H - x.shape[1])))
    for k in range(4):
        (w1, b1), (w2, b2), (w3, b3) = params[k]
        if k == 0:
            w1 = jnp.pad(w1, ((0, H - w1.shape[0]), (0, 0)))
        s = agg(h, src, dst)
        h = _mlp(h, s, w1, b1.reshape(1, H), w2, b2.reshape(1, H),
                 w3, b3.reshape(1, H))

    (wh1, bh1), (wh2, bh2) = head
    batch_p = jnp.pad(batch.astype(i32), (0, NBLK * BN - N),
                      constant_values=NG).reshape(NBLK, 1, BN)
    out = _pool_head(h, batch_p, wh1, bh1.reshape(1, H),
                     wh2, bh2.reshape(1, 1))
    return out.reshape(-1)


# speedup vs baseline: 7.5861x; 1.4893x over previous
import functools

import jax
import jax.numpy as jnp
from jax import lax
from jax.experimental import pallas as pl
from jax.experimental.pallas import tpu as pltpu
from jax.experimental.pallas import tpu_sc as plsc

N = 50000
E = 800000
H = 128
NSUB = 16
NCORE = 2
G = 128
FQ = 96

BLK = 12512
TRROWS = 32
SROWS = BLK + TRROWS
RPS = SROWS // NSUB
NPAD = 4 * BLK

EPW = E // NSUB
CH = 2000
NCH = EPW // CH
SUB = 80
NSC = CH // SUB

f32 = jnp.float32
i32 = jnp.int32


def _agg_body(t_hbm, srcr, dstr, out, aggsh, zrows, sv, dv,
              fifo_s, fifo_d, rows, gs, gd, sem0, sem1):
    c = lax.axis_index("c")
    s = lax.axis_index("s")
    lanes = lax.iota(i32, 16)

    def zf(j, _):
        for k in range(8):
            zrows[j, pl.ds(k * 16, 16)] = jnp.zeros((16,), f32)
        return 0
    lax.fori_loop(0, 8, zf, 0)

    def fire_effects(nf):
        p = nf & 1
        for g in range(FQ // 16):
            sl = pl.ds(g * 16, 16)
            gs[p, sl] = fifo_s[0, sl]
            gd[p, sl] = fifo_d[0, sl]

        @pl.when(p == 0)
        def _():
            pltpu.async_copy(t_hbm.at[gs.at[0]], rows.at[0], sem0)

        @pl.when(p == 1)
        def _():
            pltpu.async_copy(t_hbm.at[gs.at[1]], rows.at[1], sem1)

        @pl.when(nf >= 1)
        def _():
            @pl.when(p == 1)
            def _():
                pltpu.make_async_copy(t_hbm.at[gs.at[0]], rows.at[0], sem0).wait()
                pltpu.sync_copy(rows.at[0], aggsh.at[gd.at[0]], add=True)

            @pl.when(p == 0)
            def _():
                pltpu.make_async_copy(t_hbm.at[gs.at[1]], rows.at[1], sem1).wait()
                pltpu.sync_copy(rows.at[1], aggsh.at[gd.at[1]], add=True)

        for g in range(5):
            srcg = g + 6
            if srcg < 8:
                vs_ = fifo_s[0, pl.ds(srcg * 16, 16)]
                vd_ = fifo_d[0, pl.ds(srcg * 16, 16)]
            else:
                vs_ = fifo_s[1, pl.ds((srcg - 8) * 16, 16)]
                vd_ = fifo_d[1, pl.ds((srcg - 8) * 16, 16)]
            fifo_s[0, pl.ds(g * 16, 16)] = vs_
            fifo_d[0, pl.ds(g * 16, 16)] = vd_

    def drain(nf):
        @pl.when(nf >= 1)
        def _():
            pl_ = (nf - 1) & 1

            @pl.when(pl_ == 0)
            def _():
                pltpu.make_async_copy(t_hbm.at[gs.at[0]], rows.at[0], sem0).wait()
                pltpu.sync_copy(rows.at[0], aggsh.at[gd.at[0]], add=True)

            @pl.when(pl_ == 1)
            def _():
                pltpu.make_async_copy(t_hbm.at[gs.at[1]], rows.at[1], sem1).wait()
                pltpu.sync_copy(rows.at[1], aggsh.at[gd.at[1]], add=True)

    def scan_sub(cnt, nf, lo, off):
        base = jnp.full((16,), cnt, i32)
        for j in range(SUB // 16):
            sl = pl.ds(off + j * 16, 16)
            d = dv[sl]
            sr = sv[sl]
            dr = d - lo
            m = (dr >= 0) & (dr < BLK)
            csum = plsc.cumsum(m.astype(i32))
            pos = base + csum - 1
            row = (pos >= G).astype(i32)
            col = pos - row * G
            plsc.store_scatter(fifo_d, [row, col], dr, mask=m)
            plsc.store_scatter(fifo_s, [row, col], sr, mask=m)
            base = base + plsc.all_reduce_population_count(m)
        cnt = jnp.max(base)
        fired = cnt >= FQ

        @pl.when(fired)
        def _():
            fire_effects(nf)
        cnt = cnt - jnp.where(fired, FQ, 0)
        nf = nf + jnp.where(fired, 1, 0)
        return cnt, nf

    for p in range(2):
        b = c * 2 + p
        lo = b * BLK

        def zb(j, _):
            pltpu.sync_copy(zrows, aggsh.at[pl.ds(s * RPS + j * 8, 8)])
            return 0
        lax.fori_loop(0, RPS // 8, zb, 0)
        plsc.subcore_barrier()

        e0 = s * EPW

        def chunk(i, st):
            cnt, nf = st
            pltpu.sync_copy(srcr.at[pl.ds(e0 + i * CH, CH)], sv)
            pltpu.sync_copy(dstr.at[pl.ds(e0 + i * CH, CH)], dv)

            def sub(g, st):
                return scan_sub(st[0], st[1], lo, g * SUB)
            return lax.fori_loop(0, NSC, sub, (cnt, nf))
        cnt, nf = lax.fori_loop(0, NCH, chunk, (jnp.int32(0), jnp.int32(0)))

        cnt_v = jnp.full((16,), cnt, i32)
        padsrc = (s * 16 + lanes) * 8
        for k in range(FQ // 16):
            sl = pl.ds(k * 16, 16)
            padmask = (lanes + k * 16) >= cnt_v
            fifo_d[0, sl] = jnp.where(padmask, BLK + (lanes & 15), fifo_d[0, sl])
            fifo_s[0, sl] = jnp.where(padmask, padsrc, fifo_s[0, sl])

        @pl.when(cnt > 0)
        def _():
            fire_effects(nf)
        nf = nf + jnp.where(cnt > 0, 1, 0)
        drain(nf)
        plsc.subcore_barrier()

        @pl.when(s < NSUB - 1)
        def _():
            pltpu.sync_copy(aggsh.at[pl.ds(s * RPS, RPS)],
                            out.at[pl.ds(lo + s * RPS, RPS)])

        @pl.when(s == NSUB - 1)
        def _():
            pltpu.sync_copy(aggsh.at[pl.ds((NSUB - 1) * RPS, BLK - (NSUB - 1) * RPS)],
                            out.at[pl.ds(lo + (NSUB - 1) * RPS, BLK - (NSUB - 1) * RPS)])

        plsc.subcore_barrier()


_AGG_CACHE = {}


def _get_agg():
    if "agg" not in _AGG_CACHE:
        mesh = plsc.VectorSubcoreMesh(core_axis_name="c", subcore_axis_name="s",
                                      num_cores=NCORE, num_subcores=NSUB)
        _AGG_CACHE["agg"] = functools.partial(
            pl.kernel,
            out_type=jax.ShapeDtypeStruct((NPAD, H), f32),
            mesh=mesh,
            compiler_params=pltpu.CompilerParams(needs_layout_passes=False),
            scratch_types=[
                pltpu.VMEM_SHARED((SROWS, H), f32),
                pltpu.VMEM((8, H), f32),
                pltpu.VMEM((CH,), i32),
                pltpu.VMEM((CH,), i32),
                pltpu.VMEM((2, G), i32),
                pltpu.VMEM((2, G), i32),
                pltpu.VMEM((2, FQ, H), f32),
                pltpu.VMEM((2, FQ), i32),
                pltpu.VMEM((2, FQ), i32),
                pltpu.SemaphoreType.DMA,
                pltpu.SemaphoreType.DMA,
            ],
        )(_agg_body)
    return _AGG_CACHE["agg"]


BN = 2048
NBLK = (N + BN - 1) // BN


def _mlp_body(h_ref, s_ref, w1, b1, w2, b2, w3, b3, o_ref):
    u = h_ref[...] + s_ref[...]
    z = jax.nn.relu(jnp.dot(u, w1[...], preferred_element_type=f32) + b1[...])
    z = jax.nn.relu(jnp.dot(z, w2[...], preferred_element_type=f32) + b2[...])
    z = jnp.dot(z, w3[...], preferred_element_type=f32) + b3[...]
    o_ref[...] = jax.nn.relu(z)


_mlp = pl.pallas_call(
    _mlp_body,
    grid=(NBLK,),
    in_specs=[
        pl.BlockSpec((BN, H), lambda i: (i, 0)),
        pl.BlockSpec((BN, H), lambda i: (i, 0)),
        pl.BlockSpec((H, H), lambda i: (0, 0)),
        pl.BlockSpec((1, H), lambda i: (0, 0)),
        pl.BlockSpec((H, H), lambda i: (0, 0)),
        pl.BlockSpec((1, H), lambda i: (0, 0)),
        pl.BlockSpec((H, H), lambda i: (0, 0)),
        pl.BlockSpec((1, H), lambda i: (0, 0)),
    ],
    out_specs=pl.BlockSpec((BN, H), lambda i: (i, 0)),
    out_shape=jax.ShapeDtypeStruct((N, H), f32),
)

NG = 64


def _pool_head_body(h_ref, b_ref, wh1, bh1, wh2, bh2, o_ref, sacc, cacc):
    i = pl.program_id(0)

    @pl.when(i == 0)
    def _():
        sacc[...] = jnp.zeros((NG, H), f32)
        cacc[...] = jnp.zeros((NG, 1), f32)

    bv = b_ref[0]
    oh = (bv == lax.broadcasted_iota(i32, (NG, BN), 0)).astype(f32)
    rowid = lax.broadcasted_iota(i32, (BN, H), 0) + i * BN
    hv = jnp.where(rowid < N, h_ref[...], 0.0)
    sacc[...] += jnp.dot(oh, hv, preferred_element_type=f32,
                         precision=lax.Precision.HIGHEST)
    cacc[...] += jnp.dot(oh, jnp.ones((BN, 1), f32), preferred_element_type=f32)

    @pl.when(i == NBLK - 1)
    def _():
        hg = sacc[...] / jnp.maximum(cacc[...], 1.0)
        y = jax.nn.relu(jnp.dot(hg, wh1[...], preferred_element_type=f32)
                        + bh1[...])
        o_ref[...] = jnp.dot(y, wh2[...], preferred_element_type=f32) + bh2[...]


_pool_head = pl.pallas_call(
    _pool_head_body,
    grid=(NBLK,),
    in_specs=[
        pl.BlockSpec((BN, H), lambda i: (i, 0)),
        pl.BlockSpec((1, 1, BN), lambda i: (i, 0, 0)),
        pl.BlockSpec((H, H), lambda i: (0, 0)),
        pl.BlockSpec((1, H), lambda i: (0, 0)),
        pl.BlockSpec((H, 1), lambda i: (0, 0)),
        pl.BlockSpec((1, 1), lambda i: (0, 0)),
    ],
    out_specs=pl.BlockSpec((NG, 1), lambda i: (0, 0)),
    out_shape=jax.ShapeDtypeStruct((NG, 1), f32),
    scratch_shapes=[
        pltpu.VMEM((NG, H), f32),
        pltpu.VMEM((NG, 1), f32),
    ],
)


def kernel(x, edge_index, batch, params, head):
    src = edge_index[0].astype(i32)
    dst = edge_index[1].astype(i32)

    agg = _get_agg()
    h = jnp.pad(x.astype(f32), ((0, 0), (0, H - x.shape[1])))
    for k in range(4):
        (w1, b1), (w2, b2), (w3, b3) = params[k]
        if k == 0:
            w1 = jnp.pad(w1, ((0, H - w1.shape[0]), (0, 0)))
        s = agg(h, src, dst)
        h = _mlp(h, s, w1, b1.reshape(1, H), w2, b2.reshape(1, H),
                 w3, b3.reshape(1, H))

    (wh1, bh1), (wh2, bh2) = head
    batch_p = jnp.pad(batch.astype(i32), (0, NBLK * BN - N),
                      constant_values=NG).reshape(NBLK, 1, BN)
    out = _pool_head(h, batch_p, wh1, bh1.reshape(1, H),
                     wh2, bh2.reshape(1, 1))
    return out.reshape(-1)
